# Initial kernel scaffold; baseline (speedup 1.0000x reference)
import jax
import jax.numpy as jnp
from jax.experimental import pallas as pl


def kernel(x, adj, W0, a0, W1, a1, W2, a2, W3, a3, W_out, a_out):
    raise NotImplementedError("write your pallas kernel here")


# speedup vs baseline: 1.4842x; 1.4842x over previous
import jax
import jax.numpy as jnp
from jax.experimental import pallas as pl
from jax.experimental.pallas import tpu as pltpu

ALPHA = 0.2
NEG = -9e15


def _proj1_kernel(x_ref, W_ref, A2_ref, Wh_ref, er_ref):
    xb = x_ref[...]
    nheads = W_ref.shape[0]
    cols = []
    for h in range(nheads):
        Wh = jnp.dot(xb, W_ref[h], preferred_element_type=jnp.float32)
        Wh_ref[h] = Wh
        cols.append(jnp.dot(Wh, A2_ref[h], preferred_element_type=jnp.float32))
    er_ref[...] = jnp.concatenate(cols, axis=1)


def _attn1_kernel(adj_ref, Wh_ref, A1_ref, erT_ref, Wo_ref, a2o_ref,
                  Whout_ref, er2_ref):
    i = pl.program_id(0)
    br = adj_ref.shape[0]
    nheads = Wh_ref.shape[0]
    adjb = adj_ref[...]
    acc = None
    for h in range(nheads):
        Whh = Wh_ref[h]
        Whb = Wh_ref[h, pl.ds(i * br, br), :]
        el = jnp.dot(Whb, A1_ref[h], preferred_element_type=jnp.float32)
        e = el + erT_ref[h][None, :]
        e = jnp.where(e > 0, e, ALPHA * e)
        e = jnp.where(adjb > 0, e, NEG)
        m = jnp.max(e, axis=1, keepdims=True)
        p = jnp.exp(e - m)
        s = jnp.sum(p, axis=1, keepdims=True)
        att = p / s
        hp = jnp.dot(att, Whh, preferred_element_type=jnp.float32)
        hp = jnp.where(hp > 0, hp, jnp.exp(hp) - 1.0)
        part = jnp.dot(hp, Wo_ref[h], preferred_element_type=jnp.float32)
        acc = part if acc is None else acc + part
    Whout_ref[...] = acc
    er2_ref[...] = jnp.dot(acc, a2o_ref[...],
                           preferred_element_type=jnp.float32)


def _attn2_kernel(adj_ref, Whout_ref, a1o_ref, er2T_ref, out_ref):
    i = pl.program_id(0)
    br = adj_ref.shape[0]
    adjb = adj_ref[...]
    Whob = Whout_ref[pl.ds(i * br, br), :]
    el = jnp.dot(Whob, a1o_ref[...], preferred_element_type=jnp.float32)
    e = el + er2T_ref[...]
    e = jnp.where(e > 0, e, ALPHA * e)
    e = jnp.where(adjb > 0, e, NEG)
    m = jnp.max(e, axis=1, keepdims=True)
    p = jnp.exp(e - m)
    s = jnp.sum(p, axis=1, keepdims=True)
    att = p / s
    out_ref[...] = jnp.dot(att, Whout_ref[...],
                           preferred_element_type=jnp.float32)


def kernel(x, adj, W0, a0, W1, a1, W2, a2, W3, a3, W_out, a_out):
    n, nfeat = x.shape
    nhid = W0.shape[1]
    nheads = 4
    nclass = W_out.shape[1]
    ncp = 128 * ((nclass + 127) // 128)
    br = min(256, n)
    nblk = n // br

    Ws = jnp.stack([W0, W1, W2, W3])
    A1 = jnp.stack([a0[:nhid], a1[:nhid], a2[:nhid], a3[:nhid]])
    A2 = jnp.stack([a0[nhid:], a1[nhid:], a2[nhid:], a3[nhid:]])
    Wo_p = jnp.zeros((nheads, nhid, ncp), jnp.float32).at[:, :, :nclass].set(
        W_out.reshape(nheads, nhid, nclass))
    a1o = jnp.zeros((ncp, 1), jnp.float32).at[:nclass].set(a_out[:nclass])
    a2o = jnp.zeros((ncp, 1), jnp.float32).at[:nclass].set(a_out[nclass:])

    full = lambda shape: pl.BlockSpec(shape, lambda i: (0,) * len(shape))

    Wh, er = pl.pallas_call(
        _proj1_kernel,
        grid=(nblk,),
        in_specs=[
            pl.BlockSpec((br, nfeat), lambda i: (i, 0)),
            full(Ws.shape),
            full(A1.shape),
        ],
        out_specs=[
            pl.BlockSpec((nheads, br, nhid), lambda i: (0, i, 0)),
            pl.BlockSpec((br, nheads), lambda i: (i, 0)),
        ],
        out_shape=[
            jax.ShapeDtypeStruct((nheads, n, nhid), jnp.float32),
            jax.ShapeDtypeStruct((n, nheads), jnp.float32),
        ],
    )(x, Ws, A2)

    erT = er.T

    Wh_out, er2 = pl.pallas_call(
        _attn1_kernel,
        grid=(nblk,),
        in_specs=[
            pl.BlockSpec((br, n), lambda i: (i, 0)),
            full(Wh.shape),
            full(A1.shape),
            full(erT.shape),
            full(Wo_p.shape),
            full(a2o.shape),
        ],
        out_specs=[
            pl.BlockSpec((br, ncp), lambda i: (i, 0)),
            pl.BlockSpec((br, 1), lambda i: (i, 0)),
        ],
        out_shape=[
            jax.ShapeDtypeStruct((n, ncp), jnp.float32),
            jax.ShapeDtypeStruct((n, 1), jnp.float32),
        ],
    )(adj, Wh, A1, erT, Wo_p, a2o)

    er2T = er2.reshape(1, n)

    out = pl.pallas_call(
        _attn2_kernel,
        grid=(nblk,),
        in_specs=[
            pl.BlockSpec((br, n), lambda i: (i, 0)),
            full(Wh_out.shape),
            full(a1o.shape),
            full(er2T.shape),
        ],
        out_specs=pl.BlockSpec((br, ncp), lambda i: (i, 0)),
        out_shape=jax.ShapeDtypeStruct((n, ncp), jnp.float32),
    )(adj, Wh_out, a1o, er2T)

    return out[:, :nclass]
